# Initial kernel scaffold; baseline (speedup 1.0000x reference)
import jax
import jax.numpy as jnp
from jax.experimental import pallas as pl


def kernel(values, times, var_ids, mask, vp_w, vp_b, time_freq, var_emb, mix_r, mix_i, mix_j, mix_k, mix_b, time_codes, var_codes, event_codes, pt_r, pt_i, pt_j, pt_k, pt_b, pv_r, pv_i, pv_j, pv_k, pv_b, pe_r, pe_i, pe_j, pe_k, pe_b, up_r, up_i, up_j, up_k, up_b, ln_g, ln_b):
    raise NotImplementedError("write your pallas kernel here")


# speedup vs baseline: 10.3935x; 10.3935x over previous
import jax
import jax.numpy as jnp
from jax.experimental import pallas as pl

B = 8
N = 2048
D = 256
Q = 64
NV = 128
KT = 64
KV = 128
KE = 512
TK = 4
NL = 2
TAU = 0.5


def _quat_wt(r, i, j, k):
    w = jnp.concatenate([
        jnp.concatenate([r, -i, -j, -k], 1),
        jnp.concatenate([i, r, -k, j], 1),
        jnp.concatenate([j, k, r, -i], 1),
        jnp.concatenate([k, -j, i, r], 1),
    ], 0)
    return w.T


def _hamilton(p, q):
    c = Q
    pr, pi, pj, pk = p[:, :c], p[:, c:2 * c], p[:, 2 * c:3 * c], p[:, 3 * c:]
    qr, qi, qj, qk = q[:, :c], q[:, c:2 * c], q[:, 2 * c:3 * c], q[:, 3 * c:]
    return jnp.concatenate([
        pr * qr - pi * qi - pj * qj - pk * qk,
        pr * qi + pi * qr + pj * qk - pk * qj,
        pr * qj - pi * qk + pj * qr + pk * qi,
        pr * qk + pi * qj - pj * qi + pk * qr,
    ], 1)


def _topk4(logits, k):
    iota = jax.lax.broadcasted_iota(jnp.int32, logits.shape, 1)
    v = logits
    tvs, tis = [], []
    for _ in range(TK):
        tv = jnp.max(v, axis=1, keepdims=True)
        ti = jnp.min(jnp.where(v == tv, iota, k), axis=1, keepdims=True)
        v = jnp.where(iota == ti, -1e30, v)
        tvs.append(tv)
        tis.append(ti)
    return jnp.concatenate(tvs, 1), jnp.concatenate(tis, 1)


def _route(q, q_bf, mask_col, codes, wt, bias):
    k = codes.shape[0]
    logits = jax.lax.dot_general(
        q_bf, codes.astype(jnp.bfloat16), (((1,), (1,)), ((), ())),
        preferred_element_type=jnp.float32) * (1.0 / 16.0)
    tv, ti = _topk4(logits, k)
    e = jnp.exp((tv - tv[:, :1]) * (1.0 / TAU))
    w = (e / jnp.sum(e, axis=1, keepdims=True)) * mask_col
    iota_k = jax.lax.broadcasted_iota(jnp.int32, (N, k), 1)
    wrt = jnp.zeros((N, k), jnp.float32)
    for m in range(TK):
        wrt = wrt + w[:, m:m + 1] * (iota_k == ti[:, m:m + 1]).astype(jnp.float32)
    wagg = wrt * mask_col
    proto = jax.lax.dot_general(
        wagg, q, (((0,), (0,)), ((), ())), preferred_element_type=jnp.float32,
        precision=jax.lax.Precision.HIGHEST)
    ws = jnp.sum(wagg, axis=0)[:, None]
    plocal = proto / jnp.maximum(ws, 1e-3)
    blend = jnp.clip(ws / (ws + 0.5), 0.0, 1.0)
    pb = blend * plocal + (1.0 - blend) * codes
    pb = jnp.dot(pb.astype(jnp.bfloat16), wt.astype(jnp.bfloat16),
                 preferred_element_type=jnp.float32) + bias
    gathered = jnp.dot(wrt, pb, preferred_element_type=jnp.float32,
                       precision=jax.lax.Precision.HIGHEST)
    return _hamilton(q, gathered)


def _q_layernorm(x, g, b):
    outs = []
    for c in range(4):
        xc = x[:, c * Q:(c + 1) * Q]
        mu = jnp.mean(xc, axis=1, keepdims=True)
        var = jnp.mean((xc - mu) ** 2, axis=1, keepdims=True)
        outs.append(g[c][None, :] * (xc - mu) / jnp.sqrt(var + 1e-5) + b[c][None, :])
    return jnp.concatenate(outs, 1)


def _fwd_kernel(values_ref, times_ref, var_ids_ref, mask_ref,
                vp_wa_ref, vp_wb_ref, vp_b_ref, time_freq_ref, var_emb_ref,
                mix_wt_ref, mix_b_ref,
                time_codes_ref, var_codes_ref, event_codes_ref,
                pt_wt_ref, pt_b_ref, pv_wt_ref, pv_b_ref,
                pe_wt_ref, pe_b_ref, up_wt_ref, up_b_ref,
                ln_g_ref, ln_b_ref, out_ref):
    vals = values_ref[0, 0, :]
    times = times_ref[0, 0, :]
    ids = var_ids_ref[0, 0, :]
    mask = mask_ref[0, 0, :]
    mask_col = mask[:, None]

    def _bf(x):
        return x.astype(jnp.bfloat16).astype(jnp.float32)

    q_r = (_bf(vals * mask)[:, None] * _bf(vp_wa_ref[0, :])[None, :]
           + _bf(mask)[:, None] * _bf(vp_wb_ref[0, :])[None, :]) \
        + vp_b_ref[0, :][None, :]
    tp = times[:, None] * time_freq_ref[0, :][None, :]
    iota_nv = jax.lax.broadcasted_iota(jnp.int32, (N, NV), 1)
    onehot = (iota_nv == ids[:, None]).astype(jnp.float32)
    emb = jnp.dot(onehot, var_emb_ref[:, :], preferred_element_type=jnp.float32,
                  precision=jax.lax.Precision.HIGHEST)
    q = jnp.concatenate([q_r, jnp.sin(tp), jnp.cos(tp), emb], 1)
    q = jnp.dot(q.astype(jnp.bfloat16), mix_wt_ref[:, :].astype(jnp.bfloat16),
                preferred_element_type=jnp.float32) + mix_b_ref[0, :][None, :]

    for l in range(NL):
        q_bf = q.astype(jnp.bfloat16)
        acc = _route(q, q_bf, mask_col, time_codes_ref[:, :],
                     pt_wt_ref[l], pt_b_ref[l][None, :])
        acc = acc + _route(q, q_bf, mask_col, var_codes_ref[:, :],
                           pv_wt_ref[l], pv_b_ref[l][None, :])
        acc = acc + _route(q, q_bf, mask_col, event_codes_ref[:, :],
                           pe_wt_ref[l], pe_b_ref[l][None, :])
        msg = jnp.dot(acc.astype(jnp.bfloat16),
                      up_wt_ref[l].astype(jnp.bfloat16),
                      preferred_element_type=jnp.float32) + up_b_ref[l][None, :]
        q = _q_layernorm(q + msg, ln_g_ref[l], ln_b_ref[l])

    out_ref[0, :, :] = q


def _full(shape):
    nd = len(shape)
    return pl.BlockSpec(shape, lambda b, _nd=nd: (0,) * _nd)


def kernel(values, times, var_ids, mask, vp_w, vp_b, time_freq, var_emb,
           mix_r, mix_i, mix_j, mix_k, mix_b, time_codes, var_codes,
           event_codes, pt_r, pt_i, pt_j, pt_k, pt_b, pv_r, pv_i, pv_j, pv_k,
           pv_b, pe_r, pe_i, pe_j, pe_k, pe_b, up_r, up_i, up_j, up_k, up_b,
           ln_g, ln_b):
    mix_wt = _quat_wt(mix_r, mix_i, mix_j, mix_k)
    pt_wt = jax.vmap(_quat_wt)(pt_r, pt_i, pt_j, pt_k)
    pv_wt = jax.vmap(_quat_wt)(pv_r, pv_i, pv_j, pv_k)
    pe_wt = jax.vmap(_quat_wt)(pe_r, pe_i, pe_j, pe_k)
    up_wt = jax.vmap(_quat_wt)(up_r, up_i, up_j, up_k)

    v3 = values.reshape(B, 1, N)
    t3 = times.reshape(B, 1, N)
    i3 = var_ids.astype(jnp.int32).reshape(B, 1, N)
    m3 = mask.reshape(B, 1, N)
    vp_wa = vp_w[:, 0].reshape(1, Q)
    vp_wb = vp_w[:, 1].reshape(1, Q)

    operands = [
        v3, t3, i3, m3,
        vp_wa, vp_wb, vp_b.reshape(1, Q), time_freq.reshape(1, Q), var_emb,
        mix_wt, mix_b.reshape(1, D),
        time_codes, var_codes, event_codes,
        pt_wt, pt_b, pv_wt, pv_b, pe_wt, pe_b, up_wt, up_b,
        ln_g, ln_b,
    ]

    batch_spec = pl.BlockSpec((1, 1, N), lambda b: (b, 0, 0))
    in_specs = [batch_spec] * 4 + [_full(op.shape) for op in operands[4:]]

    return pl.pallas_call(
        _fwd_kernel,
        grid=(B,),
        in_specs=in_specs,
        out_specs=pl.BlockSpec((1, N, D), lambda b: (b, 0, 0)),
        out_shape=jax.ShapeDtypeStruct((B, N, D), jnp.float32),
    )(*operands)


# speedup vs baseline: 13.5213x; 1.3009x over previous
import jax
import jax.numpy as jnp
from jax.experimental import pallas as pl

B = 8
N = 2048
D = 256
Q = 64
NV = 128
KT = 64
KV = 128
KE = 512
TK = 4
NL = 2
TAU = 0.5


def _quat_wt(r, i, j, k):
    w = jnp.concatenate([
        jnp.concatenate([r, -i, -j, -k], 1),
        jnp.concatenate([i, r, -k, j], 1),
        jnp.concatenate([j, k, r, -i], 1),
        jnp.concatenate([k, -j, i, r], 1),
    ], 0)
    return w.T


def _hamilton(p, q):
    c = Q
    pr, pi, pj, pk = p[:, :c], p[:, c:2 * c], p[:, 2 * c:3 * c], p[:, 3 * c:]
    qr, qi, qj, qk = q[:, :c], q[:, c:2 * c], q[:, 2 * c:3 * c], q[:, 3 * c:]
    return jnp.concatenate([
        pr * qr - pi * qi - pj * qj - pk * qk,
        pr * qi + pi * qr + pj * qk - pk * qj,
        pr * qj - pi * qk + pj * qr + pk * qi,
        pr * qk + pi * qj - pj * qi + pk * qr,
    ], 1)


def _topk4(logits, k):
    iota = jax.lax.broadcasted_iota(jnp.int32, logits.shape, 1)
    v = logits
    tvs, tis = [], []
    for _ in range(TK):
        tv = jnp.max(v, axis=1, keepdims=True)
        ti = jnp.min(jnp.where(v == tv, iota, k), axis=1, keepdims=True)
        v = jnp.where(iota == ti, -1e30, v)
        tvs.append(tv)
        tis.append(ti)
    return jnp.concatenate(tvs, 1), jnp.concatenate(tis, 1)


def _routing_matrix(logits, k):
    iota = jax.lax.broadcasted_iota(jnp.int32, (N, k), 1)
    v = logits
    wrt = jnp.zeros((N, k), jnp.float32)
    wsum = jnp.zeros((N, 1), jnp.float32)
    tv0 = None
    for m in range(TK):
        tv = jnp.max(v, axis=1, keepdims=True)
        if m == 0:
            tv0 = tv
            e = jnp.ones((N, 1), jnp.float32)
        else:
            e = jnp.exp((tv - tv0) * (1.0 / TAU))
        ti = jnp.min(jnp.where(v == tv, iota, k), axis=1, keepdims=True)
        first = iota == ti
        wrt = jnp.where(first, jnp.broadcast_to(e, (N, k)), wrt)
        v = jnp.where(first, -1e30, v)
        wsum = wsum + e
    return wrt / wsum


def _layer(q, q_bf, mask_col, codes_all, codes_all_bf, wts, biases, segs):
    logits = jax.lax.dot_general(
        q_bf, codes_all_bf, (((1,), (1,)), ((), ())),
        preferred_element_type=jnp.float32) * (1.0 / 16.0)
    blocks = []
    for (lo, hi) in segs:
        blocks.append(_routing_matrix(logits[:, lo:hi], hi - lo))
    wrt = jnp.concatenate(blocks, 1) * mask_col
    wagg = wrt * mask_col
    proto = jax.lax.dot_general(
        wagg, q, (((0,), (0,)), ((), ())), preferred_element_type=jnp.float32,
        precision=jax.lax.Precision.HIGHEST)
    ws = jnp.sum(wagg, axis=0)[:, None]
    plocal = proto / jnp.maximum(ws, 1e-3)
    blend = jnp.clip(ws / (ws + 0.5), 0.0, 1.0)
    pb = blend * plocal + (1.0 - blend) * codes_all
    pbl = []
    for (lo, hi), wt, bias in zip(segs, wts, biases):
        pbl.append(jnp.dot(pb[lo:hi].astype(jnp.bfloat16),
                           wt.astype(jnp.bfloat16),
                           preferred_element_type=jnp.float32) + bias)
    pb_all = jnp.concatenate(pbl, 0)
    gathered = jnp.dot(wrt, pb_all, preferred_element_type=jnp.float32,
                       precision=jax.lax.Precision.HIGHEST)
    return _hamilton(q, gathered)


def _q_layernorm(x, g, b):
    outs = []
    for c in range(4):
        xc = x[:, c * Q:(c + 1) * Q]
        mu = jnp.mean(xc, axis=1, keepdims=True)
        var = jnp.mean((xc - mu) ** 2, axis=1, keepdims=True)
        outs.append(g[c][None, :] * (xc - mu) / jnp.sqrt(var + 1e-5) + b[c][None, :])
    return jnp.concatenate(outs, 1)


def _fwd_kernel(values_ref, times_ref, var_ids_ref, mask_ref,
                vp_wa_ref, vp_wb_ref, vp_b_ref, time_freq_ref, var_emb_ref,
                mix_wt_ref, mix_b_ref,
                codes_all_ref, codes_all_bf_ref,
                pt_wt_ref, pt_b_ref, pv_wt_ref, pv_b_ref,
                pe_wt_ref, pe_b_ref, up_wt_ref, up_b_ref,
                ln_g_ref, ln_b_ref, out_ref):
    vals = values_ref[0, 0, :]
    times = times_ref[0, 0, :]
    ids = var_ids_ref[0, 0, :]
    mask = mask_ref[0, 0, :]
    mask_col = mask[:, None]

    def _bf(x):
        return x.astype(jnp.bfloat16).astype(jnp.float32)

    q_r = (_bf(vals * mask)[:, None] * _bf(vp_wa_ref[0, :])[None, :]
           + _bf(mask)[:, None] * _bf(vp_wb_ref[0, :])[None, :]) \
        + vp_b_ref[0, :][None, :]
    tp = times[:, None] * time_freq_ref[0, :][None, :]
    iota_nv = jax.lax.broadcasted_iota(jnp.int32, (N, NV), 1)
    onehot = (iota_nv == ids[:, None]).astype(jnp.float32)
    emb = jnp.dot(onehot, var_emb_ref[:, :], preferred_element_type=jnp.float32,
                  precision=jax.lax.Precision.HIGHEST)
    q = jnp.concatenate([q_r, jnp.sin(tp), jnp.cos(tp), emb], 1)
    q = jnp.dot(q.astype(jnp.bfloat16), mix_wt_ref[:, :].astype(jnp.bfloat16),
                preferred_element_type=jnp.float32) + mix_b_ref[0, :][None, :]

    segs = [(0, KV), (KV, KV + KE), (KV + KE, KV + KE + KT)]
    codes_all = codes_all_ref[:, :]
    codes_all_bf = codes_all_bf_ref[:, :]
    for l in range(NL):
        q_bf = q.astype(jnp.bfloat16)
        acc = _layer(q, q_bf, mask_col, codes_all, codes_all_bf,
                     (pv_wt_ref[l], pe_wt_ref[l], pt_wt_ref[l]),
                     (pv_b_ref[l][None, :], pe_b_ref[l][None, :],
                      pt_b_ref[l][None, :]),
                     segs)
        msg = jnp.dot(acc.astype(jnp.bfloat16),
                      up_wt_ref[l].astype(jnp.bfloat16),
                      preferred_element_type=jnp.float32) + up_b_ref[l][None, :]
        q = _q_layernorm(q + msg, ln_g_ref[l], ln_b_ref[l])

    out_ref[0, :, :] = q


def _full(shape):
    nd = len(shape)
    return pl.BlockSpec(shape, lambda b, _nd=nd: (0,) * _nd)


def kernel(values, times, var_ids, mask, vp_w, vp_b, time_freq, var_emb,
           mix_r, mix_i, mix_j, mix_k, mix_b, time_codes, var_codes,
           event_codes, pt_r, pt_i, pt_j, pt_k, pt_b, pv_r, pv_i, pv_j, pv_k,
           pv_b, pe_r, pe_i, pe_j, pe_k, pe_b, up_r, up_i, up_j, up_k, up_b,
           ln_g, ln_b):
    mix_wt = _quat_wt(mix_r, mix_i, mix_j, mix_k)
    pt_wt = jax.vmap(_quat_wt)(pt_r, pt_i, pt_j, pt_k)
    pv_wt = jax.vmap(_quat_wt)(pv_r, pv_i, pv_j, pv_k)
    pe_wt = jax.vmap(_quat_wt)(pe_r, pe_i, pe_j, pe_k)
    up_wt = jax.vmap(_quat_wt)(up_r, up_i, up_j, up_k)

    v3 = values.reshape(B, 1, N)
    t3 = times.reshape(B, 1, N)
    i3 = var_ids.astype(jnp.int32).reshape(B, 1, N)
    m3 = mask.reshape(B, 1, N)
    vp_wa = vp_w[:, 0].reshape(1, Q)
    vp_wb = vp_w[:, 1].reshape(1, Q)

    codes_all = jnp.concatenate([var_codes, event_codes, time_codes], 0)
    operands = [
        v3, t3, i3, m3,
        vp_wa, vp_wb, vp_b.reshape(1, Q), time_freq.reshape(1, Q), var_emb,
        mix_wt, mix_b.reshape(1, D),
        codes_all, codes_all.astype(jnp.bfloat16),
        pt_wt, pt_b, pv_wt, pv_b, pe_wt, pe_b, up_wt, up_b,
        ln_g, ln_b,
    ]

    batch_spec = pl.BlockSpec((1, 1, N), lambda b: (b, 0, 0))
    in_specs = [batch_spec] * 4 + [_full(op.shape) for op in operands[4:]]

    return pl.pallas_call(
        _fwd_kernel,
        grid=(B,),
        in_specs=in_specs,
        out_specs=pl.BlockSpec((1, N, D), lambda b: (b, 0, 0)),
        out_shape=jax.ShapeDtypeStruct((B, N, D), jnp.float32),
    )(*operands)
